# baseline (speedup 1.0000x reference)
import functools

import jax
import jax.numpy as jnp
from jax import lax
from jax.experimental import pallas as pl
from jax.experimental.pallas import tpu as pltpu
from jax.experimental.pallas import tpu_sc as plsc

N = 4096
E = 131072
IN_SIZE = 128
OUT_SIZE = 4
BE = 2048

NC = 2
NS = 16
L = 16
NW = NC * NS
SLAB = 128
ER = E // SLAB

TBL = N * N + E
DUMMY_KEY = N * N
ROWS_D = N + 128
DUMMY_ROW = N

CH = E // NW
NSLAB = CH // SLAB
CAND = 128

_mesh = plsc.VectorSubcoreMesh(
    core_axis_name="c", subcore_axis_name="s", num_cores=NC, num_subcores=NS)


def _ew_body(xi_ref, xj_ref, eij_ref, w_ref, b_ref, out_ref):
    q = xj_ref[...] + eij_ref[...]
    w1 = w_ref[:IN_SIZE, :]
    w2 = w_ref[IN_SIZE:, :]
    h = (jnp.dot(q, w1, preferred_element_type=jnp.float32)
         + jnp.dot(xi_ref[...], w2, preferred_element_type=jnp.float32)
         + b_ref[...])
    out_ref[...] = jnp.exp(jnp.tanh(h))


def _compute_ew(x_i, x_j, e_ij, W, b):
    return pl.pallas_call(
        _ew_body,
        grid=(E // BE,),
        in_specs=[
            pl.BlockSpec((BE, IN_SIZE), lambda i: (i, 0)),
            pl.BlockSpec((BE, IN_SIZE), lambda i: (i, 0)),
            pl.BlockSpec((BE, IN_SIZE), lambda i: (i, 0)),
            pl.BlockSpec((2 * IN_SIZE, OUT_SIZE), lambda i: (0, 0)),
            pl.BlockSpec((OUT_SIZE,), lambda i: (0,)),
        ],
        out_specs=pl.BlockSpec((BE, OUT_SIZE), lambda i: (i, 0)),
        out_shape=jax.ShapeDtypeStruct((E, OUT_SIZE), jnp.float32),
    )(x_i, x_j, e_ij, W, b)


@functools.partial(
    pl.kernel,
    mesh=_mesh,
    out_type=jax.ShapeDtypeStruct((TBL,), jnp.int32),
    scratch_types=[
        pltpu.VMEM((NSLAB, SLAB), jnp.int32),
        pltpu.VMEM((NSLAB, SLAB), jnp.int32),
        pltpu.SemaphoreType.DMA,
    ],
)
def _dedup_a(keys2d, tbl1, k_v, id_v, sem):
    c = lax.axis_index("c")
    s = lax.axis_index("s")
    wid = s * NC + c
    base = wid * CH
    pltpu.sync_copy(keys2d.at[pl.ds(wid * NSLAB, NSLAB)], k_v)
    iota = lax.iota(jnp.int32, L)

    @pl.loop(0, CH // L)
    def _build(i):
        j = i // (SLAB // L)
        t = i % (SLAB // L)
        id_v[j, pl.ds(t * L, L)] = base + i * L + iota

    @pl.loop(0, NSLAB // 16)
    def _sc(jj):
        ds = [pltpu.async_copy(id_v.at[jj * 16 + u],
                               tbl1.at[k_v.at[jj * 16 + u]], sem)
              for u in range(16)]
        for d in ds:
            d.wait()


@functools.partial(
    pl.kernel,
    mesh=_mesh,
    out_type=[jax.ShapeDtypeStruct((ER, SLAB), jnp.int32),
              jax.ShapeDtypeStruct((TBL,), jnp.int32)],
    scratch_types=[
        pltpu.VMEM((NSLAB, SLAB), jnp.int32),
        pltpu.VMEM((NSLAB, SLAB), jnp.int32),
        pltpu.VMEM((NSLAB, SLAB), jnp.int32),
        pltpu.VMEM((NSLAB, SLAB), jnp.int32),
        pltpu.SemaphoreType.DMA,
    ],
)
def _dedup_b(keys2d, tbl1, t1_hbm, tbl2, k_v, t_v, id_v, k2_v, sem):
    c = lax.axis_index("c")
    s = lax.axis_index("s")
    wid = s * NC + c
    base = wid * CH
    pltpu.sync_copy(keys2d.at[pl.ds(wid * NSLAB, NSLAB)], k_v)
    iota = lax.iota(jnp.int32, L)

    @pl.loop(0, NSLAB // 16)
    def _g(jj):
        ds = [pltpu.async_copy(tbl1.at[k_v.at[jj * 16 + u]],
                               t_v.at[jj * 16 + u], sem)
              for u in range(16)]
        for d in ds:
            d.wait()

    pltpu.sync_copy(t_v, t1_hbm.at[pl.ds(wid * NSLAB, NSLAB)])

    @pl.loop(0, CH // L)
    def _mask(i):
        j = i // (SLAB // L)
        t = i % (SLAB // L)
        tv = t_v[j, pl.ds(t * L, L)]
        iv = base + i * L + iota
        id_v[j, pl.ds(t * L, L)] = iv
        kv = k_v[j, pl.ds(t * L, L)]
        k2_v[j, pl.ds(t * L, L)] = jnp.where(iv > tv, kv, DUMMY_KEY + iv)

    @pl.loop(0, NSLAB // 16)
    def _sc(jj):
        ds = [pltpu.async_copy(id_v.at[jj * 16 + u],
                               tbl2.at[k2_v.at[jj * 16 + u]], sem)
              for u in range(16)]
        for d in ds:
            d.wait()


@functools.partial(
    pl.kernel,
    mesh=_mesh,
    out_type=jax.ShapeDtypeStruct((ER, SLAB), jnp.int32),
    scratch_types=[
        pltpu.VMEM((NSLAB, SLAB), jnp.int32),
        pltpu.VMEM((NSLAB, SLAB), jnp.int32),
        pltpu.VMEM((NSLAB, SLAB), jnp.int32),
        pltpu.VMEM((NSLAB, SLAB), jnp.int32),
        pltpu.VMEM((NSLAB, SLAB), jnp.int32),
        pltpu.SemaphoreType.DMA,
    ],
)
def _dedup_c(keys2d, keysflat, t1_hbm, tbl2, rep_hbm,
             k_v, t1_v, t2_v, c2_v, kk_v, sem):
    c = lax.axis_index("c")
    s = lax.axis_index("s")
    wid = s * NC + c
    pltpu.sync_copy(keys2d.at[pl.ds(wid * NSLAB, NSLAB)], k_v)
    pltpu.sync_copy(t1_hbm.at[pl.ds(wid * NSLAB, NSLAB)], t1_v)

    @pl.loop(0, NSLAB // 16)
    def _g2(jj):
        ds = [pltpu.async_copy(tbl2.at[k_v.at[jj * 16 + u]],
                               t2_v.at[jj * 16 + u], sem)
              for u in range(16)]
        for d in ds:
            d.wait()

    iota = lax.iota(jnp.int32, L)
    base = wid * CH

    @pl.loop(0, CH // L)
    def _clamp(i):
        j = i // (SLAB // L)
        t = i % (SLAB // L)
        tv = t2_v[j, pl.ds(t * L, L)]
        t1 = t1_v[j, pl.ds(t * L, L)]
        iv = base + i * L + iota
        cand = (tv > t1) & (tv >= 0) & (tv < E)
        c2_v[j, pl.ds(t * L, L)] = jnp.where(cand, tv, iv)

    @pl.loop(0, NSLAB // 16)
    def _gk(jj):
        ds = [pltpu.async_copy(keysflat.at[c2_v.at[jj * 16 + u]],
                               kk_v.at[jj * 16 + u], sem)
              for u in range(16)]
        for d in ds:
            d.wait()

    @pl.loop(0, CH // L)
    def _comb(i):
        j = i // (SLAB // L)
        t = i % (SLAB // L)
        kv = k_v[j, pl.ds(t * L, L)]
        kk = kk_v[j, pl.ds(t * L, L)]
        t1 = t1_v[j, pl.ds(t * L, L)]
        t2 = t2_v[j, pl.ds(t * L, L)]
        c2 = c2_v[j, pl.ds(t * L, L)]
        valid = (t2 > t1) & (t2 >= 0) & (t2 < E) & (kk == kv)
        t1_v[j, pl.ds(t * L, L)] = jnp.where(valid, c2, t1)

    pltpu.sync_copy(t1_v, rep_hbm.at[pl.ds(wid * NSLAB, NSLAB)])


@functools.partial(
    pl.kernel,
    mesh=_mesh,
    out_type=[jax.ShapeDtypeStruct((NC * ROWS_D,), jnp.float32)
              for _ in range(OUT_SIZE)],
    scratch_types=[
        [pltpu.VMEM_SHARED((ROWS_D,), jnp.float32) for _ in range(OUT_SIZE)],
        pltpu.VMEM((NSLAB, SLAB), jnp.int32),
        pltpu.VMEM((NSLAB, SLAB), jnp.int32),
        pltpu.VMEM((NSLAB, SLAB), jnp.int32),
        [pltpu.VMEM((CH,), jnp.float32) for _ in range(OUT_SIZE)],
        pltpu.SemaphoreType.DMA,
    ],
)
def _denom(rows2d, rep2d, ew0, ew1, ew2, ew3, z_hbm,
           dp0, dp1, dp2, dp3, dsh, r_v, rep_v, ri_v, ew_v, sem):
    c = lax.axis_index("c")
    s = lax.axis_index("s")
    wid = s * NC + c
    base = wid * CH
    ew_hbm = (ew0, ew1, ew2, ew3)
    dp_hbm = (dp0, dp1, dp2, dp3)

    @pl.when(s == 0)
    def _():
        for k in range(OUT_SIZE):
            pltpu.sync_copy(z_hbm, dsh[k])

    pltpu.sync_copy(rows2d.at[pl.ds(wid * NSLAB, NSLAB)], r_v)
    pltpu.sync_copy(rep2d.at[pl.ds(wid * NSLAB, NSLAB)], rep_v)
    for k in range(OUT_SIZE):
        pltpu.sync_copy(ew_hbm[k].at[pl.ds(base, CH)], ew_v[k])
    iota = lax.iota(jnp.int32, L)

    @pl.loop(0, CH // L)
    def _m(i):
        j = i // (SLAB // L)
        t = i % (SLAB // L)
        rv = r_v[j, pl.ds(t * L, L)]
        repv = rep_v[j, pl.ds(t * L, L)]
        iv = base + i * L + iota
        ri_v[j, pl.ds(t * L, L)] = jnp.where(repv == iv, rv, DUMMY_ROW)

    plsc.subcore_barrier()

    @pl.loop(0, NSLAB)
    def _sa(j):
        ds = [pltpu.async_copy(ew_v[k].at[pl.ds(j * SLAB, SLAB)],
                               dsh[k].at[ri_v.at[j]], sem, add=True)
              for k in range(OUT_SIZE)]
        for d in ds:
            d.wait()

    plsc.subcore_barrier()

    @pl.when(s == 0)
    def _():
        for k in range(OUT_SIZE):
            pltpu.sync_copy(dsh[k], dp_hbm[k].at[pl.ds(c * ROWS_D, ROWS_D)])


def _recip_body(dp0, dp1, dp2, dp3, r0, r1, r2, r3):
    for dp_ref, r_ref in ((dp0, r0), (dp1, r1), (dp2, r2), (dp3, r3)):
        r_ref[...] = 1.0 / (dp_ref[pl.ds(0, ROWS_D)]
                            + dp_ref[pl.ds(ROWS_D, ROWS_D)])


def _recip(dps):
    return pl.pallas_call(
        _recip_body,
        out_shape=[jax.ShapeDtypeStruct((ROWS_D,), jnp.float32)
                   for _ in range(OUT_SIZE)],
    )(*dps)


@functools.partial(
    pl.kernel,
    mesh=_mesh,
    out_type=[jax.ShapeDtypeStruct((E,), jnp.float32)
              for _ in range(OUT_SIZE)],
    scratch_types=[
        pltpu.VMEM((NSLAB, SLAB), jnp.int32),
        pltpu.VMEM((NSLAB, SLAB), jnp.int32),
        [pltpu.VMEM((CH,), jnp.float32) for _ in range(OUT_SIZE)],
        [pltpu.VMEM((CH,), jnp.float32) for _ in range(OUT_SIZE)],
        pltpu.SemaphoreType.DMA,
    ],
)
def _gather_mul(rows2d, rep2d, ew0, ew1, ew2, ew3, r0, r1, r2, r3,
                o0, o1, o2, o3, r_v, rep_v, ewr_v, rg_v, sem):
    c = lax.axis_index("c")
    s = lax.axis_index("s")
    wid = s * NC + c
    base = wid * CH
    ew_hbm = (ew0, ew1, ew2, ew3)
    rr_hbm = (r0, r1, r2, r3)
    out_hbm = (o0, o1, o2, o3)

    pltpu.sync_copy(rows2d.at[pl.ds(wid * NSLAB, NSLAB)], r_v)
    pltpu.sync_copy(rep2d.at[pl.ds(wid * NSLAB, NSLAB)], rep_v)

    @pl.loop(0, NSLAB // 2)
    def _g(jj):
        ds = []
        for u in range(2):
            j = jj * 2 + u
            for k in range(OUT_SIZE):
                ds.append(pltpu.async_copy(ew_hbm[k].at[rep_v.at[j]],
                                           ewr_v[k].at[pl.ds(j * SLAB, SLAB)],
                                           sem))
                ds.append(pltpu.async_copy(rr_hbm[k].at[r_v.at[j]],
                                           rg_v[k].at[pl.ds(j * SLAB, SLAB)],
                                           sem))
        for d in ds:
            d.wait()

    @pl.loop(0, CH // L)
    def _mul(i):
        for k in range(OUT_SIZE):
            ewr_v[k][pl.ds(i * L, L)] = (ewr_v[k][pl.ds(i * L, L)]
                                         * rg_v[k][pl.ds(i * L, L)])

    for k in range(OUT_SIZE):
        pltpu.sync_copy(ewr_v[k], out_hbm[k].at[pl.ds(base, CH)])


def kernel(x_i, x_j, e_ij, adj, e_idx, W, b):
    ew = _compute_ew(x_i, x_j, e_ij, W, b)
    ew_planes = [ew[:, k] for k in range(OUT_SIZE)]

    rows2d = e_idx[0].reshape(ER, SLAB)
    keys = e_idx[0] * N + e_idx[1]
    keys2d = keys.reshape(ER, SLAB)

    tbl1 = _dedup_a(keys2d)
    t1_2d, tbl2 = _dedup_b(keys2d, tbl1)
    rep2d = _dedup_c(keys2d, keys, t1_2d, tbl2)
    z = jnp.zeros((ROWS_D,), jnp.float32)
    dps = _denom(rows2d, rep2d, *ew_planes, z)
    rs = _recip(dps)
    outs = _gather_mul(rows2d, rep2d, *ew_planes, *rs)
    return jnp.stack(outs, axis=-1)


# speedup vs baseline: 1.1728x; 1.1728x over previous
import functools

import jax
import jax.numpy as jnp
from jax import lax
from jax.experimental import pallas as pl
from jax.experimental.pallas import tpu as pltpu
from jax.experimental.pallas import tpu_sc as plsc

N = 4096
E = 131072
IN_SIZE = 128
OUT_SIZE = 4
BE = 2048

NC = 2
NS = 16
L = 16
NW = NC * NS
SLAB = 128
ER = E // SLAB

TBL = N * N + E
DUMMY_KEY = N * N
ROWS_D = N + 128
DUMMY_ROW = N

CH = E // NW
NSLAB = CH // SLAB
CAND = 128

_mesh = plsc.VectorSubcoreMesh(
    core_axis_name="c", subcore_axis_name="s", num_cores=NC, num_subcores=NS)


def _ew_body(xi_ref, xj_ref, eij_ref, w_ref, b_ref, o0, o1, o2, o3):
    q = xj_ref[...] + eij_ref[...]
    w1 = w_ref[:IN_SIZE, :]
    w2 = w_ref[IN_SIZE:, :]
    h = (jnp.dot(q, w1, preferred_element_type=jnp.float32)
         + jnp.dot(xi_ref[...], w2, preferred_element_type=jnp.float32)
         + b_ref[...])
    ew = jnp.exp(jnp.tanh(h))
    for k, o_ref in enumerate((o0, o1, o2, o3)):
        o_ref[...] = ew[:, k]


def _compute_ew(x_i, x_j, e_ij, W, b):
    return pl.pallas_call(
        _ew_body,
        grid=(E // BE,),
        in_specs=[
            pl.BlockSpec((BE, IN_SIZE), lambda i: (i, 0)),
            pl.BlockSpec((BE, IN_SIZE), lambda i: (i, 0)),
            pl.BlockSpec((BE, IN_SIZE), lambda i: (i, 0)),
            pl.BlockSpec((2 * IN_SIZE, OUT_SIZE), lambda i: (0, 0)),
            pl.BlockSpec((OUT_SIZE,), lambda i: (0,)),
        ],
        out_specs=[pl.BlockSpec((BE,), lambda i: (i,))
                   for _ in range(OUT_SIZE)],
        out_shape=[jax.ShapeDtypeStruct((E,), jnp.float32)
                   for _ in range(OUT_SIZE)],
    )(x_i, x_j, e_ij, W, b)


@functools.partial(
    pl.kernel,
    mesh=_mesh,
    out_type=jax.ShapeDtypeStruct((TBL,), jnp.int32),
    scratch_types=[
        pltpu.VMEM((NSLAB, SLAB), jnp.int32),
        pltpu.VMEM((NSLAB, SLAB), jnp.int32),
        pltpu.SemaphoreType.DMA,
    ],
)
def _dedup_a(keys2d, tbl1, k_v, id_v, sem):
    c = lax.axis_index("c")
    s = lax.axis_index("s")
    wid = s * NC + c
    base = wid * CH
    pltpu.sync_copy(keys2d.at[pl.ds(wid * NSLAB, NSLAB)], k_v)
    iota = lax.iota(jnp.int32, L)

    @pl.loop(0, CH // L)
    def _build(i):
        j = i // (SLAB // L)
        t = i % (SLAB // L)
        id_v[j, pl.ds(t * L, L)] = base + i * L + iota

    @pl.loop(0, NSLAB // 16)
    def _sc(jj):
        ds = [pltpu.async_copy(id_v.at[jj * 16 + u],
                               tbl1.at[k_v.at[jj * 16 + u]], sem)
              for u in range(16)]
        for d in ds:
            d.wait()


@functools.partial(
    pl.kernel,
    mesh=_mesh,
    out_type=[jax.ShapeDtypeStruct((ER, SLAB), jnp.int32),
              jax.ShapeDtypeStruct((TBL,), jnp.int32)],
    scratch_types=[
        pltpu.VMEM((NSLAB, SLAB), jnp.int32),
        pltpu.VMEM((NSLAB, SLAB), jnp.int32),
        pltpu.VMEM((NSLAB, SLAB), jnp.int32),
        pltpu.VMEM((NSLAB, SLAB), jnp.int32),
        pltpu.SemaphoreType.DMA,
    ],
)
def _dedup_b(keys2d, tbl1, t1_hbm, tbl2, k_v, t_v, id_v, k2_v, sem):
    c = lax.axis_index("c")
    s = lax.axis_index("s")
    wid = s * NC + c
    base = wid * CH
    pltpu.sync_copy(keys2d.at[pl.ds(wid * NSLAB, NSLAB)], k_v)
    iota = lax.iota(jnp.int32, L)

    @pl.loop(0, NSLAB // 16)
    def _g(jj):
        ds = [pltpu.async_copy(tbl1.at[k_v.at[jj * 16 + u]],
                               t_v.at[jj * 16 + u], sem)
              for u in range(16)]
        for d in ds:
            d.wait()

    pltpu.sync_copy(t_v, t1_hbm.at[pl.ds(wid * NSLAB, NSLAB)])

    @pl.loop(0, CH // L)
    def _mask(i):
        j = i // (SLAB // L)
        t = i % (SLAB // L)
        tv = t_v[j, pl.ds(t * L, L)]
        iv = base + i * L + iota
        id_v[j, pl.ds(t * L, L)] = iv
        kv = k_v[j, pl.ds(t * L, L)]
        k2_v[j, pl.ds(t * L, L)] = jnp.where(iv > tv, kv, DUMMY_KEY + iv)

    @pl.loop(0, NSLAB // 16)
    def _sc(jj):
        ds = [pltpu.async_copy(id_v.at[jj * 16 + u],
                               tbl2.at[k2_v.at[jj * 16 + u]], sem)
              for u in range(16)]
        for d in ds:
            d.wait()


@functools.partial(
    pl.kernel,
    mesh=_mesh,
    out_type=jax.ShapeDtypeStruct((ER, SLAB), jnp.int32),
    scratch_types=[
        pltpu.VMEM((NSLAB, SLAB), jnp.int32),
        pltpu.VMEM((NSLAB, SLAB), jnp.int32),
        pltpu.VMEM((NSLAB, SLAB), jnp.int32),
        pltpu.VMEM((NSLAB, SLAB), jnp.int32),
        pltpu.VMEM((NSLAB, SLAB), jnp.int32),
        pltpu.SemaphoreType.DMA,
    ],
)
def _dedup_c(keys2d, keysflat, t1_hbm, tbl2, rep_hbm,
             k_v, t1_v, t2_v, c2_v, kk_v, sem):
    c = lax.axis_index("c")
    s = lax.axis_index("s")
    wid = s * NC + c
    pltpu.sync_copy(keys2d.at[pl.ds(wid * NSLAB, NSLAB)], k_v)
    pltpu.sync_copy(t1_hbm.at[pl.ds(wid * NSLAB, NSLAB)], t1_v)

    @pl.loop(0, NSLAB // 16)
    def _g2(jj):
        ds = [pltpu.async_copy(tbl2.at[k_v.at[jj * 16 + u]],
                               t2_v.at[jj * 16 + u], sem)
              for u in range(16)]
        for d in ds:
            d.wait()

    iota = lax.iota(jnp.int32, L)
    base = wid * CH

    @pl.loop(0, CH // L)
    def _clamp(i):
        j = i // (SLAB // L)
        t = i % (SLAB // L)
        tv = t2_v[j, pl.ds(t * L, L)]
        t1 = t1_v[j, pl.ds(t * L, L)]
        iv = base + i * L + iota
        cand = (tv > t1) & (tv >= 0) & (tv < E)
        c2_v[j, pl.ds(t * L, L)] = jnp.where(cand, tv, iv)

    @pl.loop(0, NSLAB // 16)
    def _gk(jj):
        ds = [pltpu.async_copy(keysflat.at[c2_v.at[jj * 16 + u]],
                               kk_v.at[jj * 16 + u], sem)
              for u in range(16)]
        for d in ds:
            d.wait()

    @pl.loop(0, CH // L)
    def _comb(i):
        j = i // (SLAB // L)
        t = i % (SLAB // L)
        kv = k_v[j, pl.ds(t * L, L)]
        kk = kk_v[j, pl.ds(t * L, L)]
        t1 = t1_v[j, pl.ds(t * L, L)]
        t2 = t2_v[j, pl.ds(t * L, L)]
        c2 = c2_v[j, pl.ds(t * L, L)]
        valid = (t2 > t1) & (t2 >= 0) & (t2 < E) & (kk == kv)
        t1_v[j, pl.ds(t * L, L)] = jnp.where(valid, c2, t1)

    pltpu.sync_copy(t1_v, rep_hbm.at[pl.ds(wid * NSLAB, NSLAB)])


@functools.partial(
    pl.kernel,
    mesh=_mesh,
    out_type=[jax.ShapeDtypeStruct((NC * ROWS_D,), jnp.float32)
              for _ in range(OUT_SIZE)],
    scratch_types=[
        [pltpu.VMEM_SHARED((ROWS_D,), jnp.float32) for _ in range(OUT_SIZE)],
        pltpu.VMEM((NSLAB, SLAB), jnp.int32),
        pltpu.VMEM((NSLAB, SLAB), jnp.int32),
        pltpu.VMEM((NSLAB, SLAB), jnp.int32),
        [pltpu.VMEM((CH,), jnp.float32) for _ in range(OUT_SIZE)],
        pltpu.SemaphoreType.DMA,
    ],
)
def _denom(rows2d, rep2d, ew0, ew1, ew2, ew3, z_hbm,
           dp0, dp1, dp2, dp3, dsh, r_v, rep_v, ri_v, ew_v, sem):
    c = lax.axis_index("c")
    s = lax.axis_index("s")
    wid = s * NC + c
    base = wid * CH
    ew_hbm = (ew0, ew1, ew2, ew3)
    dp_hbm = (dp0, dp1, dp2, dp3)

    @pl.when(s == 0)
    def _():
        for k in range(OUT_SIZE):
            pltpu.sync_copy(z_hbm, dsh[k])

    pltpu.sync_copy(rows2d.at[pl.ds(wid * NSLAB, NSLAB)], r_v)
    pltpu.sync_copy(rep2d.at[pl.ds(wid * NSLAB, NSLAB)], rep_v)
    for k in range(OUT_SIZE):
        pltpu.sync_copy(ew_hbm[k].at[pl.ds(base, CH)], ew_v[k])
    iota = lax.iota(jnp.int32, L)

    @pl.loop(0, CH // L)
    def _m(i):
        j = i // (SLAB // L)
        t = i % (SLAB // L)
        rv = r_v[j, pl.ds(t * L, L)]
        repv = rep_v[j, pl.ds(t * L, L)]
        iv = base + i * L + iota
        ri_v[j, pl.ds(t * L, L)] = jnp.where(repv == iv, rv, DUMMY_ROW)

    plsc.subcore_barrier()

    @pl.loop(0, NSLAB)
    def _sa(j):
        ds = [pltpu.async_copy(ew_v[k].at[pl.ds(j * SLAB, SLAB)],
                               dsh[k].at[ri_v.at[j]], sem, add=True)
              for k in range(OUT_SIZE)]
        for d in ds:
            d.wait()

    plsc.subcore_barrier()

    @pl.when(s == 0)
    def _():
        for k in range(OUT_SIZE):
            pltpu.sync_copy(dsh[k], dp_hbm[k].at[pl.ds(c * ROWS_D, ROWS_D)])


def _recip_body(dp0, dp1, dp2, dp3, r0, r1, r2, r3):
    for dp_ref, r_ref in ((dp0, r0), (dp1, r1), (dp2, r2), (dp3, r3)):
        r_ref[...] = 1.0 / (dp_ref[pl.ds(0, ROWS_D)]
                            + dp_ref[pl.ds(ROWS_D, ROWS_D)])


def _recip(dps):
    return pl.pallas_call(
        _recip_body,
        out_shape=[jax.ShapeDtypeStruct((ROWS_D,), jnp.float32)
                   for _ in range(OUT_SIZE)],
    )(*dps)


@functools.partial(
    pl.kernel,
    mesh=_mesh,
    out_type=[jax.ShapeDtypeStruct((E,), jnp.float32)
              for _ in range(OUT_SIZE)],
    scratch_types=[
        pltpu.VMEM((NSLAB, SLAB), jnp.int32),
        pltpu.VMEM((NSLAB, SLAB), jnp.int32),
        [pltpu.VMEM((CH,), jnp.float32) for _ in range(OUT_SIZE)],
        [pltpu.VMEM((CH,), jnp.float32) for _ in range(OUT_SIZE)],
        pltpu.SemaphoreType.DMA,
    ],
)
def _gather_mul(rows2d, rep2d, ew0, ew1, ew2, ew3, r0, r1, r2, r3,
                o0, o1, o2, o3, r_v, rep_v, ewr_v, rg_v, sem):
    c = lax.axis_index("c")
    s = lax.axis_index("s")
    wid = s * NC + c
    base = wid * CH
    ew_hbm = (ew0, ew1, ew2, ew3)
    rr_hbm = (r0, r1, r2, r3)
    out_hbm = (o0, o1, o2, o3)

    pltpu.sync_copy(rows2d.at[pl.ds(wid * NSLAB, NSLAB)], r_v)
    pltpu.sync_copy(rep2d.at[pl.ds(wid * NSLAB, NSLAB)], rep_v)

    @pl.loop(0, NSLAB // 2)
    def _g(jj):
        ds = []
        for u in range(2):
            j = jj * 2 + u
            for k in range(OUT_SIZE):
                ds.append(pltpu.async_copy(ew_hbm[k].at[rep_v.at[j]],
                                           ewr_v[k].at[pl.ds(j * SLAB, SLAB)],
                                           sem))
                ds.append(pltpu.async_copy(rr_hbm[k].at[r_v.at[j]],
                                           rg_v[k].at[pl.ds(j * SLAB, SLAB)],
                                           sem))
        for d in ds:
            d.wait()

    @pl.loop(0, CH // L)
    def _mul(i):
        for k in range(OUT_SIZE):
            ewr_v[k][pl.ds(i * L, L)] = (ewr_v[k][pl.ds(i * L, L)]
                                         * rg_v[k][pl.ds(i * L, L)])

    for k in range(OUT_SIZE):
        pltpu.sync_copy(ewr_v[k], out_hbm[k].at[pl.ds(base, CH)])


def kernel(x_i, x_j, e_ij, adj, e_idx, W, b):
    ew_planes = _compute_ew(x_i, x_j, e_ij, W, b)

    rows2d = e_idx[0].reshape(ER, SLAB)
    keys = e_idx[0] * N + e_idx[1]
    keys2d = keys.reshape(ER, SLAB)

    tbl1 = _dedup_a(keys2d)
    t1_2d, tbl2 = _dedup_b(keys2d, tbl1)
    rep2d = _dedup_c(keys2d, keys, t1_2d, tbl2)
    z = jnp.zeros((ROWS_D,), jnp.float32)
    dps = _denom(rows2d, rep2d, *ew_planes, z)
    rs = _recip(dps)
    outs = _gather_mul(rows2d, rep2d, *ew_planes, *rs)
    return jnp.stack(outs, axis=-1)
